# baseline (speedup 1.0000x reference)
import functools

import jax
import jax.numpy as jnp
from jax import lax
from jax.experimental import pallas as pl
from jax.experimental.pallas import tpu as pltpu
from jax.experimental.pallas import tpu_sc as plsc

B = 16384
D = 128
CHUNK = 64
LANES = 16
NGROUP = D // LANES


NBUF = 3


def _body(nc, ns, uid, iid, uemb, iemb, ubias, ibias, out,
          uidx_v, iidx_v, urows, irows, ub_v, ib_v, t16, out_v, *sems):
    nw = nc * ns
    bpw = B // nw
    nchunks = bpw // CHUNK
    wid = lax.axis_index("s") * nc + lax.axis_index("c")
    base = wid * bpw
    iota = lax.iota(jnp.int32, LANES)

    hu = pltpu.async_copy(uid.at[pl.ds(base, bpw)], uidx_v, sems[0])
    hi = pltpu.async_copy(iid.at[pl.ds(base, bpw)], iidx_v, sems[1])
    hu.wait()
    hi.wait()

    def issue(c):
        par = c % NBUF
        uix = uidx_v.at[pl.ds(c * CHUNK, CHUNK)]
        iix = iidx_v.at[pl.ds(c * CHUNK, CHUNK)]
        bsl = pl.ds(c * CHUNK, CHUNK)
        return [
            pltpu.async_copy(uemb.at[uix], urows.at[par], sems[par]),
            pltpu.async_copy(iemb.at[iix], irows.at[par], sems[par]),
            pltpu.async_copy(ubias.at[uix], ub_v.at[bsl], sems[par]),
            pltpu.async_copy(ibias.at[iix], ib_v.at[bsl], sems[par]),
        ]

    def compute(c):
        par = c % NBUF
        u = urows.at[par]
        v = irows.at[par]

        @pl.loop(0, CHUNK // LANES)
        def _group(g):
            for r in range(LANES):
                b = g * LANES + r
                s = u[b, pl.ds(0, LANES)] * v[b, pl.ds(0, LANES)]
                for j in range(1, NGROUP):
                    s = s + (u[b, pl.ds(j * LANES, LANES)]
                             * v[b, pl.ds(j * LANES, LANES)])
                t16[r, :] = s
            off = pl.ds(c * CHUNK + g * LANES, LANES)
            cols = [plsc.load_gather(t16, [iota, jnp.full((LANES,), t, jnp.int32)])
                    for t in range(LANES)]
            cols.append(ub_v[off] + ib_v[off])
            while len(cols) > 1:
                cols = [a + b for a, b in zip(cols[::2], cols[1::2])] + (
                    [cols[-1]] if len(cols) & 1 else [])
            out_v[off] = cols[0]

    inflight = {c: issue(c) for c in range(min(NBUF - 1, nchunks))}
    out_handles = []
    for c in range(nchunks):
        nxt = c + NBUF - 1
        if nxt < nchunks:
            inflight[nxt] = issue(nxt)
        for h in inflight.pop(c):
            h.wait()
        compute(c)
        out_handles.append(pltpu.async_copy(
            out_v.at[pl.ds(c * CHUNK, CHUNK)],
            out.at[pl.ds(base + c * CHUNK, CHUNK)], sems[NBUF]))

    for h in out_handles:
        h.wait()


@functools.cache
def _build():
    info = plsc.get_sparse_core_info()
    nc, ns = info.num_cores, info.num_subcores
    bpw = B // (nc * ns)
    mesh = plsc.VectorSubcoreMesh(core_axis_name="c", subcore_axis_name="s")
    return pl.kernel(
        functools.partial(_body, nc, ns),
        out_type=jax.ShapeDtypeStruct((B,), jnp.float32),
        mesh=mesh,
        compiler_params=pltpu.CompilerParams(needs_layout_passes=False),
        scratch_types=[
            pltpu.VMEM((bpw,), jnp.int32),
            pltpu.VMEM((bpw,), jnp.int32),
            pltpu.VMEM((NBUF, CHUNK, D), jnp.float32),
            pltpu.VMEM((NBUF, CHUNK, D), jnp.float32),
            pltpu.VMEM((bpw,), jnp.float32),
            pltpu.VMEM((bpw,), jnp.float32),
            pltpu.VMEM((LANES, LANES), jnp.float32),
            pltpu.VMEM((bpw,), jnp.float32),
        ] + [pltpu.SemaphoreType.DMA] * (NBUF + 1),
    )


@jax.jit
def kernel(user_ids, item_ids, user_emb, item_emb, user_bias, item_bias):
    uid = user_ids.astype(jnp.int32)
    iid = item_ids.astype(jnp.int32)
    ub = user_bias.reshape(-1)
    ib = item_bias.reshape(-1)
    return _build()(uid, iid, user_emb, item_emb, ub, ib)


# speedup vs baseline: 1.0468x; 1.0468x over previous
import functools

import jax
import jax.numpy as jnp
from jax import lax
from jax.experimental import pallas as pl
from jax.experimental.pallas import tpu as pltpu
from jax.experimental.pallas import tpu_sc as plsc

B = 16384
D = 128
CHUNK = 128
LANES = 16
NGROUP = D // LANES


NBUF = 2


def _body(nc, ns, uid, iid, uemb, iemb, ubias, ibias, out,
          uidx_v, iidx_v, urows, irows, ub_v, ib_v, t16, out_v, *sems):
    nw = nc * ns
    bpw = B // nw
    nchunks = bpw // CHUNK
    wid = lax.axis_index("s") * nc + lax.axis_index("c")
    base = wid * bpw
    iota = lax.iota(jnp.int32, LANES)

    hu = pltpu.async_copy(uid.at[pl.ds(base, bpw)], uidx_v, sems[0])
    hi = pltpu.async_copy(iid.at[pl.ds(base, bpw)], iidx_v, sems[1])
    hu.wait()
    hi.wait()

    HALF = CHUNK // 2

    def issue(c):
        par = c % NBUF
        bsl = pl.ds(c * CHUNK, CHUNK)
        hs = []
        for h in range(2):
            uix = uidx_v.at[pl.ds(c * CHUNK + h * HALF, HALF)]
            iix = iidx_v.at[pl.ds(c * CHUNK + h * HALF, HALF)]
            dsl = pl.ds(h * HALF, HALF)
            hs.append(pltpu.async_copy(
                uemb.at[uix], urows.at[par, dsl], sems[par]))
            hs.append(pltpu.async_copy(
                iemb.at[iix], irows.at[par, dsl], sems[par]))
        hs.append(pltpu.async_copy(ubias.at[uidx_v.at[bsl]], ub_v.at[bsl], sems[par]))
        hs.append(pltpu.async_copy(ibias.at[iidx_v.at[bsl]], ib_v.at[bsl], sems[par]))
        return hs

    def compute(c):
        par = c % NBUF
        u = urows.at[par]
        v = irows.at[par]

        @pl.loop(0, CHUNK // LANES)
        def _group(g):
            for r in range(LANES):
                b = g * LANES + r
                s = u[b, pl.ds(0, LANES)] * v[b, pl.ds(0, LANES)]
                for j in range(1, NGROUP):
                    s = s + (u[b, pl.ds(j * LANES, LANES)]
                             * v[b, pl.ds(j * LANES, LANES)])
                t16[r, :] = s
            off = pl.ds(c * CHUNK + g * LANES, LANES)
            cols = [plsc.load_gather(t16, [iota, jnp.full((LANES,), t, jnp.int32)])
                    for t in range(LANES)]
            cols.append(ub_v[off] + ib_v[off])
            while len(cols) > 1:
                cols = [a + b for a, b in zip(cols[::2], cols[1::2])] + (
                    [cols[-1]] if len(cols) & 1 else [])
            out_v[off] = cols[0]

    inflight = {c: issue(c) for c in range(min(NBUF - 1, nchunks))}
    out_handles = []
    for c in range(nchunks):
        nxt = c + NBUF - 1
        if nxt < nchunks:
            inflight[nxt] = issue(nxt)
        for h in inflight.pop(c):
            h.wait()
        compute(c)
        out_handles.append(pltpu.async_copy(
            out_v.at[pl.ds(c * CHUNK, CHUNK)],
            out.at[pl.ds(base + c * CHUNK, CHUNK)], sems[NBUF]))

    for h in out_handles:
        h.wait()


@functools.cache
def _build():
    info = plsc.get_sparse_core_info()
    nc, ns = info.num_cores, info.num_subcores
    bpw = B // (nc * ns)
    mesh = plsc.VectorSubcoreMesh(core_axis_name="c", subcore_axis_name="s")
    return pl.kernel(
        functools.partial(_body, nc, ns),
        out_type=jax.ShapeDtypeStruct((B,), jnp.float32),
        mesh=mesh,
        compiler_params=pltpu.CompilerParams(needs_layout_passes=False),
        scratch_types=[
            pltpu.VMEM((bpw,), jnp.int32),
            pltpu.VMEM((bpw,), jnp.int32),
            pltpu.VMEM((NBUF, CHUNK, D), jnp.float32),
            pltpu.VMEM((NBUF, CHUNK, D), jnp.float32),
            pltpu.VMEM((bpw,), jnp.float32),
            pltpu.VMEM((bpw,), jnp.float32),
            pltpu.VMEM((LANES, LANES), jnp.float32),
            pltpu.VMEM((bpw,), jnp.float32),
        ] + [pltpu.SemaphoreType.DMA] * (NBUF + 1),
    )


@jax.jit
def kernel(user_ids, item_ids, user_emb, item_emb, user_bias, item_bias):
    uid = user_ids.astype(jnp.int32)
    iid = item_ids.astype(jnp.int32)
    ub = user_bias.reshape(-1)
    ib = item_bias.reshape(-1)
    return _build()(uid, iid, user_emb, item_emb, ub, ib)


# speedup vs baseline: 1.0497x; 1.0028x over previous
import functools

import jax
import jax.numpy as jnp
from jax import lax
from jax.experimental import pallas as pl
from jax.experimental.pallas import tpu as pltpu
from jax.experimental.pallas import tpu_sc as plsc

B = 16384
D = 128
CHUNK = 128
LANES = 16
NGROUP = D // LANES


NBUF = 2


def _body(nc, ns, uid, iid, uemb, iemb, ubias, ibias, out,
          uidx_v, iidx_v, urows, irows, ub_v, ib_v, t16, out_v, *sems):
    nw = nc * ns
    bpw = B // nw
    nchunks = bpw // CHUNK
    wid = lax.axis_index("s") * nc + lax.axis_index("c")
    base = wid * bpw
    iota = lax.iota(jnp.int32, LANES)

    hu = pltpu.async_copy(uid.at[pl.ds(base, bpw)], uidx_v, sems[0])
    hi = pltpu.async_copy(iid.at[pl.ds(base, bpw)], iidx_v, sems[1])
    hu.wait()
    hi.wait()

    def issue(c):
        par = c % NBUF
        uix = uidx_v.at[pl.ds(c * CHUNK, CHUNK)]
        iix = iidx_v.at[pl.ds(c * CHUNK, CHUNK)]
        bsl = pl.ds(c * CHUNK, CHUNK)
        return [
            pltpu.async_copy(uemb.at[uix], urows.at[par], sems[par]),
            pltpu.async_copy(iemb.at[iix], irows.at[par], sems[par]),
            pltpu.async_copy(ubias.at[uix], ub_v.at[bsl], sems[par]),
            pltpu.async_copy(ibias.at[iix], ib_v.at[bsl], sems[par]),
        ]

    def compute(c):
        par = c % NBUF
        u = urows.at[par]
        v = irows.at[par]

        @pl.loop(0, CHUNK // LANES)
        def _group(g):
            for r in range(LANES):
                b = g * LANES + r
                s = u[b, pl.ds(0, LANES)] * v[b, pl.ds(0, LANES)]
                for j in range(1, NGROUP):
                    s = s + (u[b, pl.ds(j * LANES, LANES)]
                             * v[b, pl.ds(j * LANES, LANES)])
                t16[r, :] = s
            off = pl.ds(c * CHUNK + g * LANES, LANES)
            cols = [plsc.load_gather(t16, [iota, jnp.full((LANES,), t, jnp.int32)])
                    for t in range(LANES)]
            cols.append(ub_v[off] + ib_v[off])
            while len(cols) > 1:
                cols = [a + b for a, b in zip(cols[::2], cols[1::2])] + (
                    [cols[-1]] if len(cols) & 1 else [])
            out_v[off] = cols[0]

    inflight = {c: issue(c) for c in range(min(NBUF - 1, nchunks))}
    out_handles = []
    for c in range(nchunks):
        nxt = c + NBUF - 1
        if nxt < nchunks:
            inflight[nxt] = issue(nxt)
        for h in inflight.pop(c):
            h.wait()
        compute(c)
        out_handles.append(pltpu.async_copy(
            out_v.at[pl.ds(c * CHUNK, CHUNK)],
            out.at[pl.ds(base + c * CHUNK, CHUNK)], sems[NBUF]))

    for h in out_handles:
        h.wait()


@functools.cache
def _build():
    info = plsc.get_sparse_core_info()
    nc, ns = info.num_cores, info.num_subcores
    bpw = B // (nc * ns)
    mesh = plsc.VectorSubcoreMesh(core_axis_name="c", subcore_axis_name="s")
    return pl.kernel(
        functools.partial(_body, nc, ns),
        out_type=jax.ShapeDtypeStruct((B,), jnp.float32),
        mesh=mesh,
        compiler_params=pltpu.CompilerParams(
            needs_layout_passes=False, disable_bounds_checks=True),
        scratch_types=[
            pltpu.VMEM((bpw,), jnp.int32),
            pltpu.VMEM((bpw,), jnp.int32),
            pltpu.VMEM((NBUF, CHUNK, D), jnp.float32),
            pltpu.VMEM((NBUF, CHUNK, D), jnp.float32),
            pltpu.VMEM((bpw,), jnp.float32),
            pltpu.VMEM((bpw,), jnp.float32),
            pltpu.VMEM((LANES, LANES), jnp.float32),
            pltpu.VMEM((bpw,), jnp.float32),
        ] + [pltpu.SemaphoreType.DMA] * (NBUF + 1),
    )


@jax.jit
def kernel(user_ids, item_ids, user_emb, item_emb, user_bias, item_bias):
    uid = user_ids.astype(jnp.int32)
    iid = item_ids.astype(jnp.int32)
    ub = user_bias.reshape(-1)
    ib = item_bias.reshape(-1)
    return _build()(uid, iid, user_emb, item_emb, ub, ib)
